# baseline (speedup 1.0000x reference)
import jax
import jax.numpy as jnp
from jax.experimental import pallas as pl
from jax.experimental.pallas import tpu as pltpu
from jax.experimental.pallas import tpu_sc as plsc

K = 32
RB = 512
TILE_G = 128
GW = 128
C1 = 128


_DN_T = (((1,), (1,)), ((), ()))


def _topk_body(pos_ref, post_ref, w1_ref, idx_ref, a_ref):
    pr = pos_ref[0]
    pc = post_ref[0]
    a_ref[0] = jax.lax.dot_general(pr, w1_ref[...], _DN_T,
                                   preferred_element_type=jnp.float32)
    dx = pr[:, 0:1] - pc[0:1, :]
    dy = pr[:, 1:2] - pc[1:2, :]
    dz = pr[:, 2:3] - pc[2:3, :]
    d = jnp.sqrt(dx * dx + dy * dy + dz * dz)
    lanes = jax.lax.broadcasted_iota(jnp.int32, d.shape, 1).astype(jnp.float32)
    big = jnp.float32(1e9)
    inf = jnp.float32(jnp.inf)
    cols = []
    for _ in range(K):
        m = jnp.min(d, axis=1, keepdims=True)
        sel = jnp.min(jnp.where(d == m, lanes, big), axis=1, keepdims=True)
        cols.append(sel)
        d = jnp.where(lanes == sel, inf, d)
    idx_ref[0] = jnp.concatenate(cols, axis=1).astype(jnp.int32)


def _topk(pos, post, w1t):
    B, N, _ = pos.shape
    return pl.pallas_call(
        _topk_body,
        grid=(B, N // RB),
        in_specs=[
            pl.BlockSpec((1, RB, 3), lambda b, r: (b, r, 0)),
            pl.BlockSpec((1, 3, N), lambda b, r: (b, 0, 0)),
            pl.BlockSpec((C1, 3), lambda b, r: (0, 0)),
        ],
        out_specs=[
            pl.BlockSpec((1, RB, K), lambda b, r: (b, r, 0)),
            pl.BlockSpec((1, RB, C1), lambda b, r: (b, r, 0)),
        ],
        out_shape=[
            jax.ShapeDtypeStruct((B, N, K), jnp.int32),
            jax.ShapeDtypeStruct((B, N, C1), jnp.float32),
        ],
        compiler_params=pltpu.CompilerParams(
            dimension_semantics=("parallel", "parallel")),
    )(pos, post, w1t)


def _sc_gather(a2, idx):
    n, k = idx.shape
    n_idx = n * k
    rows = GW // k
    mesh = plsc.VectorSubcoreMesh(core_axis_name="core",
                                  subcore_axis_name="subcore")

    @pl.kernel(out_type=jax.ShapeDtypeStruct((n_idx, C1), jnp.float32),
               mesh=mesh)
    def kern(x_hbm, i_hbm, o_hbm):
        def body(i_vmem, o_vmem):
            for j in range(rows):
                pltpu.sync_copy(x_hbm.at[i_vmem.at[j]],
                                o_vmem.at[pl.ds(j * k, k)])

        pltpu.emit_pipeline(
            body,
            grid=(n_idx // GW,),
            in_specs=[pl.BlockSpec((rows, k), lambda i: (i, 0))],
            out_specs=[pl.BlockSpec((GW, C1), lambda i: (i, 0))],
            core_axis_name=("core", "subcore"),
            dimension_semantics=(pltpu.PARALLEL,),
        )(i_hbm, o_hbm)

    return kern(a2, idx)


def _encoder_body(ag_ref, ctr_ref, b1_ref, w2_ref, b2_ref,
                  w3a_ref, w3b_ref, b3_ref, w4_ref, b4_ref, out_ref):
    tile = TILE_G * K
    ag = ag_ref[...]
    c = ctr_ref[...]
    h1 = (ag.reshape(TILE_G, K, C1) - c[:, None, :]).reshape(tile, C1)
    h1 = jnp.maximum(h1 + b1_ref[...], 0.0)
    h2 = jax.lax.dot_general(h1, w2_ref[...], _DN_T,
                             preferred_element_type=jnp.float32)
    h2 = h2 + b2_ref[...]
    m = jnp.max(h2.reshape(TILE_G, K, 256), axis=1)
    s = jax.lax.dot_general(m, w3a_ref[...], _DN_T,
                            preferred_element_type=jnp.float32)
    sb = jnp.broadcast_to(s[:, None, :], (TILE_G, K, 512)).reshape(tile, 512)
    h3 = jax.lax.dot_general(h2, w3b_ref[...], _DN_T,
                             preferred_element_type=jnp.float32)
    h3 = jnp.maximum(h3 + sb + b3_ref[...], 0.0)
    h4 = jax.lax.dot_general(h3, w4_ref[...], _DN_T,
                             preferred_element_type=jnp.float32)
    h4 = h4 + b4_ref[...]
    o = jnp.max(h4.reshape(TILE_G, K, 256), axis=1)
    out_ref[0] = o.T


def _encoder(ag, ctr, b1f, w2t, b2, w3at, w3bt, b3f, w4t, b4, B, N):
    n_groups = ctr.shape[0]
    tile = TILE_G * K
    ng = N // TILE_G
    full = lambda a: pl.BlockSpec(a.shape, lambda i: (0,) * a.ndim)
    return pl.pallas_call(
        _encoder_body,
        grid=(n_groups // TILE_G,),
        in_specs=[
            pl.BlockSpec((tile, C1), lambda i: (i, 0)),
            pl.BlockSpec((TILE_G, C1), lambda i: (i, 0)),
            full(b1f), full(w2t), full(b2),
            full(w3at), full(w3bt), full(b3f), full(w4t), full(b4),
        ],
        out_specs=pl.BlockSpec((1, 256, TILE_G),
                               lambda i: (i // ng, 0, i % ng)),
        out_shape=jax.ShapeDtypeStruct((B, 256, N), jnp.float32),
        compiler_params=pltpu.CompilerParams(
            dimension_semantics=("parallel",)),
    )(ag, ctr, b1f, w2t, b2, w3at, w3bt, b3f, w4t, b4)


def kernel(pos, W1, b1, g1, bt1, rm1, rv1, W2, b2, W3, b3, g3, bt3, rm3, rv3,
           W4, b4):
    B, N, d = pos.shape
    s1 = g1 / jnp.sqrt(rv1 + 1e-5)
    w1f = W1 * s1[:, None]
    b1f = ((b1 - rm1) * s1 + bt1)[None, :]
    s3 = g3 / jnp.sqrt(rv3 + 1e-5)
    w3f = W3 * s3[:, None]
    b3f = ((b3 - rm3) * s3 + bt3)[None, :]
    w3a = w3f[:, :256]
    w3b = w3f[:, 256:]

    post = pos.transpose(0, 2, 1)

    idxs, feats = [], []
    ags, a2s = [], []
    for b in range(B):
        idx_b, a_b = _topk(pos[b:b + 1], post[b:b + 1], w1f)
        a2_b = a_b.reshape(N, C1)
        ags.append(_sc_gather(a2_b, idx_b.reshape(N, K)))
        a2s.append(a2_b)
        idxs.append(idx_b)
    for b in range(B):
        feats.append(_encoder(ags[b], a2s[b], b1f, W2, b2[None, :],
                              w3a, w3b, b3f, W4, b4[None, :],
                              1, N))
    grouped_feat = jnp.concatenate(feats, axis=0)
    idx = jnp.concatenate(idxs, axis=0)
    return grouped_feat, idx


# speedup vs baseline: 1.0277x; 1.0277x over previous
import jax
import jax.numpy as jnp
from jax.experimental import pallas as pl
from jax.experimental.pallas import tpu as pltpu
from jax.experimental.pallas import tpu_sc as plsc

K = 32
RB = 256
TILE_G = 128
GW = 128
C1 = 128


_DN_T = (((1,), (1,)), ((), ()))


def _topk_body(pos_ref, post_ref, w1_ref, idx_ref, a_ref):
    pr = pos_ref[0]
    pc = post_ref[0]
    a_ref[0] = jax.lax.dot_general(pr, w1_ref[...], _DN_T,
                                   preferred_element_type=jnp.float32)
    dx = pr[:, 0:1] - pc[0:1, :]
    dy = pr[:, 1:2] - pc[1:2, :]
    dz = pr[:, 2:3] - pc[2:3, :]
    d = jnp.sqrt(dx * dx + dy * dy + dz * dz)
    lanes = jax.lax.broadcasted_iota(jnp.int32, d.shape, 1).astype(jnp.float32)
    big = jnp.float32(1e9)
    inf = jnp.float32(jnp.inf)
    cols = []
    for _ in range(K):
        m = jnp.min(d, axis=1, keepdims=True)
        sel = jnp.min(jnp.where(d == m, lanes, big), axis=1, keepdims=True)
        cols.append(sel)
        d = jnp.where(lanes == sel, inf, d)
    idx_ref[0] = jnp.concatenate(cols, axis=1).astype(jnp.int32)


def _topk(pos, post, w1t):
    B, N, _ = pos.shape
    return pl.pallas_call(
        _topk_body,
        grid=(B, N // RB),
        in_specs=[
            pl.BlockSpec((1, RB, 3), lambda b, r: (b, r, 0)),
            pl.BlockSpec((1, 3, N), lambda b, r: (b, 0, 0)),
            pl.BlockSpec((C1, 3), lambda b, r: (0, 0)),
        ],
        out_specs=[
            pl.BlockSpec((1, RB, K), lambda b, r: (b, r, 0)),
            pl.BlockSpec((1, RB, C1), lambda b, r: (b, r, 0)),
        ],
        out_shape=[
            jax.ShapeDtypeStruct((B, N, K), jnp.int32),
            jax.ShapeDtypeStruct((B, N, C1), jnp.float32),
        ],
        compiler_params=pltpu.CompilerParams(
            dimension_semantics=("parallel", "parallel")),
    )(pos, post, w1t)


def _sc_gather(a2, idx):
    n, k = idx.shape
    n_idx = n * k
    rows = GW // k
    mesh = plsc.VectorSubcoreMesh(core_axis_name="core",
                                  subcore_axis_name="subcore")

    @pl.kernel(out_type=jax.ShapeDtypeStruct((n_idx, C1), jnp.float32),
               mesh=mesh)
    def kern(x_hbm, i_hbm, o_hbm):
        def body(i_vmem, o_vmem):
            for j in range(rows):
                pltpu.sync_copy(x_hbm.at[i_vmem.at[j]],
                                o_vmem.at[pl.ds(j * k, k)])

        pltpu.emit_pipeline(
            body,
            grid=(n_idx // GW,),
            in_specs=[pl.BlockSpec((rows, k), lambda i: (i, 0))],
            out_specs=[pl.BlockSpec((GW, C1), lambda i: (i, 0))],
            core_axis_name=("core", "subcore"),
            dimension_semantics=(pltpu.PARALLEL,),
        )(i_hbm, o_hbm)

    return kern(a2, idx)


def _encoder_body(ag_ref, ctr_ref, b1_ref, w2_ref, b2_ref,
                  w3a_ref, w3b_ref, b3_ref, w4_ref, b4_ref, out_ref):
    tile = TILE_G * K
    ag = ag_ref[...]
    c = ctr_ref[...]
    h1 = (ag.reshape(TILE_G, K, C1) - c[:, None, :]).reshape(tile, C1)
    h1 = jnp.maximum(h1 + b1_ref[...], 0.0)
    h2 = jax.lax.dot_general(h1, w2_ref[...], _DN_T,
                             preferred_element_type=jnp.float32)
    h2 = h2 + b2_ref[...]
    m = jnp.max(h2.reshape(TILE_G, K, 256), axis=1)
    s = jax.lax.dot_general(m, w3a_ref[...], _DN_T,
                            preferred_element_type=jnp.float32)
    sb = jnp.broadcast_to(s[:, None, :], (TILE_G, K, 512)).reshape(tile, 512)
    h3 = jax.lax.dot_general(h2, w3b_ref[...], _DN_T,
                             preferred_element_type=jnp.float32)
    h3 = jnp.maximum(h3 + sb + b3_ref[...], 0.0)
    h4 = jax.lax.dot_general(h3, w4_ref[...], _DN_T,
                             preferred_element_type=jnp.float32)
    h4 = h4 + b4_ref[...]
    o = jnp.max(h4.reshape(TILE_G, K, 256), axis=1)
    out_ref[0] = o.T


def _encoder(ag, ctr, b1f, w2t, b2, w3at, w3bt, b3f, w4t, b4, B, N):
    n_groups = ctr.shape[0]
    tile = TILE_G * K
    ng = N // TILE_G
    full = lambda a: pl.BlockSpec(a.shape, lambda i: (0,) * a.ndim)
    return pl.pallas_call(
        _encoder_body,
        grid=(n_groups // TILE_G,),
        in_specs=[
            pl.BlockSpec((tile, C1), lambda i: (i, 0)),
            pl.BlockSpec((TILE_G, C1), lambda i: (i, 0)),
            full(b1f), full(w2t), full(b2),
            full(w3at), full(w3bt), full(b3f), full(w4t), full(b4),
        ],
        out_specs=pl.BlockSpec((1, 256, TILE_G),
                               lambda i: (i // ng, 0, i % ng)),
        out_shape=jax.ShapeDtypeStruct((B, 256, N), jnp.float32),
        compiler_params=pltpu.CompilerParams(
            dimension_semantics=("parallel",)),
    )(ag, ctr, b1f, w2t, b2, w3at, w3bt, b3f, w4t, b4)


def kernel(pos, W1, b1, g1, bt1, rm1, rv1, W2, b2, W3, b3, g3, bt3, rm3, rv3,
           W4, b4):
    B, N, d = pos.shape
    s1 = g1 / jnp.sqrt(rv1 + 1e-5)
    w1f = W1 * s1[:, None]
    b1f = ((b1 - rm1) * s1 + bt1)[None, :]
    s3 = g3 / jnp.sqrt(rv3 + 1e-5)
    w3f = W3 * s3[:, None]
    b3f = ((b3 - rm3) * s3 + bt3)[None, :]
    w3a = w3f[:, :256]
    w3b = w3f[:, 256:]

    post = pos.transpose(0, 2, 1)

    idxs, feats = [], []
    ags, a2s = [], []
    for b in range(B):
        idx_b, a_b = _topk(pos[b:b + 1], post[b:b + 1], w1f)
        a2_b = a_b.reshape(N, C1)
        ags.append(_sc_gather(a2_b, idx_b.reshape(N, K)))
        a2s.append(a2_b)
        idxs.append(idx_b)
    for b in range(B):
        feats.append(_encoder(ags[b], a2s[b], b1f, W2, b2[None, :],
                              w3a, w3b, b3f, W4, b4[None, :],
                              1, N))
    grouped_feat = jnp.concatenate(feats, axis=0)
    idx = jnp.concatenate(idxs, axis=0)
    return grouped_feat, idx


# speedup vs baseline: 1.0309x; 1.0031x over previous
import jax
import jax.numpy as jnp
from jax.experimental import pallas as pl
from jax.experimental.pallas import tpu as pltpu
from jax.experimental.pallas import tpu_sc as plsc

K = 32
RB = 256
TILE_G = 128
GW = 128
C1 = 128


_DN_T = (((1,), (1,)), ((), ()))


def _topk_body(pos_ref, post_ref, w1_ref, s1_ref, idx_ref, a_ref):
    pr = pos_ref[0]
    pc = post_ref[0]
    a_ref[...] = jax.lax.dot_general(pr, w1_ref[...], _DN_T,
                                     preferred_element_type=jnp.float32
                                     ) * s1_ref[...]
    dx = pr[:, 0:1] - pc[0:1, :]
    dy = pr[:, 1:2] - pc[1:2, :]
    dz = pr[:, 2:3] - pc[2:3, :]
    d = jnp.sqrt(dx * dx + dy * dy + dz * dz)
    lanes = jax.lax.broadcasted_iota(jnp.int32, d.shape, 1).astype(jnp.float32)
    big = jnp.float32(1e9)
    inf = jnp.float32(jnp.inf)
    cols = []
    for _ in range(K):
        m = jnp.min(d, axis=1, keepdims=True)
        sel = jnp.min(jnp.where(d == m, lanes, big), axis=1, keepdims=True)
        cols.append(sel)
        d = jnp.where(lanes == sel, inf, d)
    idx_ref[...] = jnp.concatenate(cols, axis=1).astype(jnp.int32)


def _topk(pos, post, w1, s1):
    _, N, _ = pos.shape
    return pl.pallas_call(
        _topk_body,
        grid=(N // RB,),
        in_specs=[
            pl.BlockSpec((1, RB, 3), lambda r: (0, r, 0)),
            pl.BlockSpec((1, 3, N), lambda r: (0, 0, 0)),
            pl.BlockSpec((C1, 3), lambda r: (0, 0)),
            pl.BlockSpec((1, C1), lambda r: (0, 0)),
        ],
        out_specs=[
            pl.BlockSpec((RB, K), lambda r: (r, 0)),
            pl.BlockSpec((RB, C1), lambda r: (r, 0)),
        ],
        out_shape=[
            jax.ShapeDtypeStruct((N, K), jnp.int32),
            jax.ShapeDtypeStruct((N, C1), jnp.float32),
        ],
        compiler_params=pltpu.CompilerParams(
            dimension_semantics=("parallel",)),
    )(pos, post, w1, s1)


def _sc_gather(a2, idx):
    n, k = idx.shape
    n_idx = n * k
    rows = GW // k
    mesh = plsc.VectorSubcoreMesh(core_axis_name="core",
                                  subcore_axis_name="subcore")

    @pl.kernel(out_type=jax.ShapeDtypeStruct((n_idx, C1), jnp.float32),
               mesh=mesh)
    def kern(x_hbm, i_hbm, o_hbm):
        def body(i_vmem, o_vmem):
            for j in range(rows):
                pltpu.sync_copy(x_hbm.at[i_vmem.at[j]],
                                o_vmem.at[pl.ds(j * k, k)])

        pltpu.emit_pipeline(
            body,
            grid=(n_idx // GW,),
            in_specs=[pl.BlockSpec((rows, k), lambda i: (i, 0))],
            out_specs=[pl.BlockSpec((GW, C1), lambda i: (i, 0))],
            core_axis_name=("core", "subcore"),
            dimension_semantics=(pltpu.PARALLEL,),
        )(i_hbm, o_hbm)

    return kern(a2, idx)


def _encoder_body(ag_ref, ctr_ref, b1_ref, w2_ref, b2_ref,
                  w3a_ref, w3b_ref, b3_ref, w4_ref, b4_ref, out_ref):
    tile = TILE_G * K
    ag = ag_ref[...]
    c = ctr_ref[...]
    h1 = (ag.reshape(TILE_G, K, C1) - c[:, None, :]).reshape(tile, C1)
    h1 = jnp.maximum(h1 + b1_ref[...], 0.0)
    h2 = jax.lax.dot_general(h1, w2_ref[...], _DN_T,
                             preferred_element_type=jnp.float32)
    h2 = h2 + b2_ref[...]
    m = jnp.max(h2.reshape(TILE_G, K, 256), axis=1)
    s = jax.lax.dot_general(m, w3a_ref[...], _DN_T,
                            preferred_element_type=jnp.float32)
    sb = jnp.broadcast_to(s[:, None, :], (TILE_G, K, 512)).reshape(tile, 512)
    h3 = jax.lax.dot_general(h2, w3b_ref[...], _DN_T,
                             preferred_element_type=jnp.float32)
    h3 = jnp.maximum(h3 + sb + b3_ref[...], 0.0)
    h4 = jax.lax.dot_general(h3, w4_ref[...], _DN_T,
                             preferred_element_type=jnp.float32)
    h4 = h4 + b4_ref[...]
    o = jnp.max(h4.reshape(TILE_G, K, 256), axis=1)
    out_ref[0] = o.T


def _encoder(ag, ctr, b1f, w2t, b2, w3at, w3bt, b3f, w4t, b4, B, N):
    n_groups = ctr.shape[0]
    tile = TILE_G * K
    ng = N // TILE_G
    full = lambda a: pl.BlockSpec(a.shape, lambda i: (0,) * a.ndim)
    return pl.pallas_call(
        _encoder_body,
        grid=(n_groups // TILE_G,),
        in_specs=[
            pl.BlockSpec((tile, C1), lambda i: (i, 0)),
            pl.BlockSpec((TILE_G, C1), lambda i: (i, 0)),
            full(b1f), full(w2t), full(b2),
            full(w3at), full(w3bt), full(b3f), full(w4t), full(b4),
        ],
        out_specs=pl.BlockSpec((1, 256, TILE_G),
                               lambda i: (i // ng, 0, i % ng)),
        out_shape=jax.ShapeDtypeStruct((B, 256, N), jnp.float32),
        compiler_params=pltpu.CompilerParams(
            dimension_semantics=("parallel",)),
    )(ag, ctr, b1f, w2t, b2, w3at, w3bt, b3f, w4t, b4)


def kernel(pos, W1, b1, g1, bt1, rm1, rv1, W2, b2, W3, b3, g3, bt3, rm3, rv3,
           W4, b4):
    B, N, d = pos.shape
    s1 = (g1 / jnp.sqrt(rv1 + 1e-5))[None, :]
    b1f = ((b1 - rm1) * s1[0] + bt1)[None, :]
    s3 = g3 / jnp.sqrt(rv3 + 1e-5)
    w3f = W3 * s3[:, None]
    b3f = ((b3 - rm3) * s3 + bt3)[None, :]
    w3a = w3f[:, :256]
    w3b = w3f[:, 256:]

    post = pos.transpose(0, 2, 1)

    idxs, feats = [], []
    ags, a2s = [], []
    for b in range(B):
        idx_b, a2_b = _topk(pos[b:b + 1], post[b:b + 1], W1, s1)
        ags.append(_sc_gather(a2_b, idx_b))
        a2s.append(a2_b)
        idxs.append(idx_b[None])
    for b in range(B):
        feats.append(_encoder(ags[b], a2s[b], b1f, W2, b2[None, :],
                              w3a, w3b, b3f, W4, b4[None, :],
                              1, N))
    grouped_feat = jnp.concatenate(feats, axis=0)
    idx = jnp.concatenate(idxs, axis=0)
    return grouped_feat, idx
